# baseline (speedup 1.0000x reference)
import functools
import jax
import jax.numpy as jnp
from jax import lax
from jax.experimental import pallas as pl
from jax.experimental.pallas import tpu as pltpu
from jax.experimental.pallas import tpu_sc as plsc

D = 64
SCALE = 8.0
G = 128
GPC = 4
C = G * GPC
SK = C + 8
NC = 2
NS = 16
NW = NC * NS


def _body(nb, na, x_hbm, t_hbm, out_hbm, idx_v, rows_v, tbuf, gsem, osem):
    wid = lax.axis_index("s") * NC + lax.axis_index("c")
    tiles_per_b = na // C
    per_w = (nb * tiles_per_b) // NW
    skew = [(lax.iota(jnp.int32, 16) + j * 16) * SK for j in range(D // 16)]

    def tile_body(t, carry):
        tile = t * NW + wid
        b = tile // tiles_per_b
        ac = tile % tiles_per_b
        pltpu.sync_copy(x_hbm.at[b, pl.ds(ac * GPC, GPC)], idx_v)
        copies = [
            pltpu.async_copy(
                t_hbm.at[idx_v.at[j]],
                rows_v.at[pl.ds(j * G, G)],
                gsem,
            )
            for j in range(GPC)
        ]
        for cp in copies:
            cp.wait()

        def row_body(k, carry2):
            r = rows_v.at[k]
            for j in range(D // 16):
                v = r[pl.ds(j * 16, 16)] * SCALE
                plsc.store_scatter(tbuf, [skew[j] + k], v)
            return carry2

        lax.fori_loop(0, C, row_body, 0, unroll=2)

        def f_body(f, carry3):
            pltpu.async_copy(
                tbuf.at[pl.ds(f * SK, C)],
                out_hbm.at[b, f, pl.ds(ac * C, C)],
                osem,
            )
            return carry3

        lax.fori_loop(0, D, f_body, 0)
        pltpu.make_async_copy(t_hbm.at[pl.ds(0, C)], rows_v, osem).wait()
        return carry

    lax.fori_loop(0, per_w, tile_body, 0)


@functools.partial(jax.jit, static_argnames=("nb", "na"))
def _sc_lookup(xr, tlin, nb, na):
    mesh = plsc.VectorSubcoreMesh(core_axis_name="c", subcore_axis_name="s")
    k = pl.kernel(
        functools.partial(_body, nb, na),
        mesh=mesh,
        compiler_params=pltpu.CompilerParams(
            use_tc_tiling_on_sc=False, needs_layout_passes=False
        ),
        out_type=jax.ShapeDtypeStruct((nb, D, na), jnp.float32),
        scratch_types=[
            pltpu.VMEM((GPC, G), jnp.int32),
            pltpu.VMEM((C, D), jnp.float32),
            pltpu.VMEM((D * SK,), jnp.float32),
            pltpu.SemaphoreType.DMA,
            pltpu.SemaphoreType.DMA,
        ],
    )
    return k(xr, tlin)


def kernel(x, table):
    vocab = table.shape[0]
    na, nb = x.shape
    t2 = jax.lax.optimization_barrier(jnp.reshape(table, (vocab // 2, 2 * D)))
    tlin = jnp.reshape(t2, (vocab, D))
    xr = jnp.reshape(x.T, (nb, na // G, G))
    outp = _sc_lookup(xr, tlin, nb, na)
    return outp.transpose(2, 0, 1)
